# baseline (speedup 1.0000x reference)
import functools

import numpy as np
import jax
import jax.numpy as jnp
from jax import lax
from jax.experimental import pallas as pl
from jax.experimental.pallas import tpu as pltpu
from jax.experimental.pallas import tpu_sc as plsc

N = 10000
E = 320000
HID = 128
NF = 128
NI = 6
NG = 50
NGP = 64
CUTOFF = 10.0
LOG2 = float(np.log(2.0))
DELTA = CUTOFF / (NG - 1)
COEFF = -0.5 / DELTA**2

NCORES = 2
NSUB = 16
NW = NCORES * NSUB
EPW = E // NW
K = 40
NB = EPW // K
EP = 320512
NCH = 5
CHB = NB // NCH
WCH = 632
WCH_LAST = N - (NSUB - 1) * WCH

_sc_mesh = plsc.VectorSubcoreMesh(core_axis_name="c", subcore_axis_name="s")
_sc_params = pltpu.CompilerParams(needs_layout_passes=False)


def _ssp(x):
    return jnp.maximum(x, 0.0) + jnp.log(1.0 + jnp.exp(-jnp.abs(x))) - LOG2


@functools.partial(
    pl.kernel,
    mesh=_sc_mesh,
    compiler_params=_sc_params,
    out_type=jax.ShapeDtypeStruct((NW, NB, K), jnp.float32),
    scratch_types=[
        pltpu.VMEM((N,), jnp.float32),
        pltpu.VMEM((N,), jnp.float32),
        pltpu.VMEM((N,), jnp.float32),
        pltpu.VMEM((NB, K), jnp.int32),
        pltpu.VMEM((NB, K), jnp.int32),
        pltpu.VMEM((NB, K), jnp.float32),
    ],
)
def _d2(posx, posy, posz, row3d, col3d, d2_out, px, py, pz, ridx, cidx, d2v):
    cid = lax.axis_index("c")
    sid = lax.axis_index("s")
    wid = cid * NSUB + sid
    pltpu.sync_copy(posx, px)
    pltpu.sync_copy(posy, py)
    pltpu.sync_copy(posz, pz)
    pltpu.sync_copy(row3d.at[wid], ridx)
    pltpu.sync_copy(col3d.at[wid], cidx)

    @pl.loop(0, NB)
    def _row(j):
        for k in range(K // 16):
            sl = pl.ds(k * 16, 16)
            r = ridx[j, sl]
            c = cidx[j, sl]
            dx = plsc.load_gather(px, [r]) - plsc.load_gather(px, [c])
            dy = plsc.load_gather(py, [r]) - plsc.load_gather(py, [c])
            dz = plsc.load_gather(pz, [r]) - plsc.load_gather(pz, [c])
            d2v[j, sl] = dx * dx + dy * dy + dz * dz

    pltpu.sync_copy(d2v, d2_out.at[wid])


_BR = 8
_EBLK = _BR * 128


def _wgen_body(d2_ref, w1_ref, w2_ref, b2_ref, out_ref):
    d2 = d2_ref[...]
    d = jnp.sqrt(d2)
    cm = 0.5 * (jnp.cos(d * (np.pi / CUTOFF)) + 1.0)
    w1 = w1_ref[...]
    w2 = w2_ref[...]
    b2 = b2_ref[...]
    g_iota = lax.broadcasted_iota(jnp.int32, (NGP, 128), 0)
    offs = g_iota.astype(jnp.float32) * DELTA
    is_last = g_iota == NGP - 1
    for sb in range(_BR):
        drow = d[sb:sb + 1, :]
        crow = cm[sb:sb + 1, :]
        ea = jnp.exp(COEFF * (drow - offs) ** 2)
        ea = jnp.where(is_last, 1.0, ea)
        h1 = jnp.dot(w1, ea, preferred_element_type=jnp.float32)
        h1 = _ssp(h1) * crow
        w = lax.dot_general(h1, w2, (((0,), (0,)), ((), ())),
                            preferred_element_type=jnp.float32)
        out_ref[sb * 128:(sb + 1) * 128, :] = w + b2


def _wgen(d2p, w1tp_i, w2_i, b2r_i):
    nsteps = (EP // 128) // _BR
    return pl.pallas_call(
        _wgen_body,
        grid=(nsteps,),
        in_specs=[
            pl.BlockSpec((_BR, 128), lambda e: (e, 0)),
            pl.BlockSpec((NF, NGP), lambda e: (0, 0)),
            pl.BlockSpec((NF, NF), lambda e: (0, 0)),
            pl.BlockSpec((1, NF), lambda e: (0, 0)),
        ],
        out_specs=pl.BlockSpec((_EBLK, NF), lambda e: (e, 0)),
        out_shape=jax.ShapeDtypeStruct((EP, NF), jnp.float32),
    )(d2p, w1tp_i, w2_i, b2r_i)


def _make_b(i):
    @functools.partial(
        pl.kernel,
        mesh=_sc_mesh,
        compiler_params=_sc_params,
        out_type=jax.ShapeDtypeStruct((NCORES, N, HID), jnp.float32),
        scratch_types=[
            pltpu.VMEM((CHB, K), jnp.int32),
            pltpu.VMEM((CHB, K), jnp.int32),
            pltpu.VMEM((K, HID), jnp.float32),
            pltpu.VMEM((K, HID), jnp.float32),
            pltpu.VMEM((K, HID), jnp.float32),
            pltpu.VMEM((K, HID), jnp.float32),
            pltpu.VMEM((K, HID), jnp.float32),
            pltpu.VMEM((K, HID), jnp.float32),
            pltpu.VMEM_SHARED((N, HID), jnp.float32),
            pltpu.SemaphoreType.DMA,
            pltpu.SemaphoreType.DMA,
            pltpu.SemaphoreType.DMA,
            pltpu.SemaphoreType.DMA,
            pltpu.SemaphoreType.DMA,
            pltpu.SemaphoreType.DMA,
            pltpu.SemaphoreType.DMA,
            pltpu.SemaphoreType.DMA,
            pltpu.SemaphoreType.DMA,
        ],
    )
    def _b(xi, w_i, row4d, col4d, zer, out, ridx, cidx,
           rows0, w0, rows1, w1, rows2, w2, agg,
           sg0, sg1, sg2, sw0, sw1, sw2, ss0, ss1, ss2):
        cid = lax.axis_index("c")
        sid = lax.axis_index("s")
        wid = cid * NSUB + sid
        r0 = sid * WCH

        def _agg_chunk_copy(src_of, dst_of):
            @pl.when(sid < NSUB - 1)
            def _a():
                pltpu.sync_copy(src_of(pl.ds(r0, WCH)), dst_of(pl.ds(r0, WCH)))

            @pl.when(sid == NSUB - 1)
            def _c():
                pltpu.sync_copy(src_of(pl.ds(r0, WCH_LAST)),
                                dst_of(pl.ds(r0, WCH_LAST)))

        _agg_chunk_copy(lambda s: zer.at[s], lambda s: agg.at[s])
        plsc.subcore_barrier()
        ebase = wid * EPW

        def _mul(rows_, wv_):
            @plsc.parallel_loop(0, K, unroll=4)
            def _mj(j):
                for v in range(HID // 16):
                    sl = pl.ds(v * 16, 16)
                    rows_[j, sl] = rows_[j, sl] * wv_[j, sl]

        slots = ((rows0, w0, sg0, sw0, ss0),
                 (rows1, w1, sg1, sw1, ss1),
                 (rows2, w2, sg2, sw2, ss2))

        def _drain(sem, buf):
            pltpu.make_async_copy(zer.at[pl.ds(0, K)], buf, sem).wait()

        def _process(bb, s):
            rows_, w_, sg_, sw_, ss_ = slots[s]
            _drain(sg_, rows_)
            _drain(sw_, w_)
            _mul(rows_, w_)
            pltpu.async_copy(rows_, agg.at[cidx.at[bb]], ss_, add=True)

        @pl.loop(0, NCH)
        def _chunk(c):
            base = c * CHB

            def _issue(bb, s):
                rows_, w_, sg_, sw_, _ = slots[s]
                pltpu.async_copy(xi.at[ridx.at[bb]], rows_, sg_)
                pltpu.async_copy(
                    w_i.at[pl.ds(ebase + (base + bb) * K, K)], w_, sw_)

            @pl.when(c > 0)
            def _dc():
                _drain(ss0, rows0)
                _drain(ss1, rows1)

            pltpu.sync_copy(row4d.at[wid, c], ridx)
            pltpu.sync_copy(col4d.at[wid, c], cidx)
            _issue(0, 0)
            _issue(1, 1)

            @pl.loop(0, (CHB - 2) // 3)
            def _triple(t):
                bb = 3 * t

                @pl.when(c + t > 0)
                def _d2s():
                    _drain(ss2, rows2)

                _issue(bb + 2, 2)
                _process(bb, 0)
                _process(bb + 1, 1)
                _drain(ss0, rows0)
                _issue(bb + 3, 0)
                _process(bb + 2, 2)
                _drain(ss1, rows1)
                _issue(bb + 4, 1)

            _process(CHB - 2, 0)
            _process(CHB - 1, 1)

        _drain(ss0, rows0)
        _drain(ss1, rows1)
        _drain(ss2, rows2)

        plsc.subcore_barrier()
        _agg_chunk_copy(lambda s: agg.at[s], lambda s: out.at[cid, s])

    return _b


_b_calls = [_make_b(i) for i in range(NI)]


_BN = 1000


def _mm_body(a_ref, b_ref, o_ref):
    o_ref[...] = jnp.dot(a_ref[...], b_ref[...],
                         preferred_element_type=jnp.float32)


def _mm(a, b):
    return pl.pallas_call(
        _mm_body,
        grid=(N // _BN,),
        in_specs=[
            pl.BlockSpec((_BN, HID), lambda n: (n, 0)),
            pl.BlockSpec((HID, NF), lambda n: (0, 0)),
        ],
        out_specs=pl.BlockSpec((_BN, NF), lambda n: (n, 0)),
        out_shape=jax.ShapeDtypeStruct((N, NF), jnp.float32),
    )(a, b)


def _c_body(parts_ref, x_ref, l2w_ref, l2b_ref, lw_ref, lb_ref, l1n_ref,
            xn_ref, xin_ref):
    agg = parts_ref[0] + parts_ref[1]
    t = jnp.dot(agg, l2w_ref[...], preferred_element_type=jnp.float32)
    t = _ssp(t + l2b_ref[...])
    t = jnp.dot(t, lw_ref[...], preferred_element_type=jnp.float32)
    xn = x_ref[...] + t + lb_ref[...]
    xn_ref[...] = xn
    xin_ref[...] = jnp.dot(xn, l1n_ref[...],
                           preferred_element_type=jnp.float32)


def _c(parts, x, l2w, l2b, lw, lb, l1n):
    return pl.pallas_call(
        _c_body,
        grid=(N // _BN,),
        in_specs=[
            pl.BlockSpec((NCORES, _BN, HID), lambda n: (0, n, 0)),
            pl.BlockSpec((_BN, HID), lambda n: (n, 0)),
            pl.BlockSpec((HID, NF), lambda n: (0, 0)),
            pl.BlockSpec((1, NF), lambda n: (0, 0)),
            pl.BlockSpec((NF, HID), lambda n: (0, 0)),
            pl.BlockSpec((1, HID), lambda n: (0, 0)),
            pl.BlockSpec((HID, NF), lambda n: (0, 0)),
        ],
        out_specs=[
            pl.BlockSpec((_BN, HID), lambda n: (n, 0)),
            pl.BlockSpec((_BN, NF), lambda n: (n, 0)),
        ],
        out_shape=[
            jax.ShapeDtypeStruct((N, HID), jnp.float32),
            jax.ShapeDtypeStruct((N, NF), jnp.float32),
        ],
    )(parts, x, l2w, l2b, lw, lb, l1n)


def kernel(h, pos, edge_index, edge_attr, mlp_w1, mlp_b1, mlp_w2, mlp_b2,
           lin1_w, lin2_w, lin2_b, lin_w, lin_b):
    row3d = edge_index[0].reshape(NW, NB, K)
    col3d = edge_index[1].reshape(NW, NB, K)
    row4d = edge_index[0].reshape(NW, NCH, CHB, K)
    col4d = edge_index[1].reshape(NW, NCH, CHB, K)
    posT = pos.T

    d2_3d = _d2(posT[0], posT[1], posT[2], row3d, col3d)
    d2p = jnp.pad(d2_3d.reshape(E), (0, EP - E)).reshape(EP // 128, 128)

    w1t = jnp.transpose(mlp_w1, (0, 2, 1))
    w1tp = jnp.zeros((NI, NF, NGP), jnp.float32)
    w1tp = w1tp.at[:, :, :NG].set(w1t).at[:, :, NGP - 1].set(mlp_b1)
    b2r = mlp_b2[:, None, :]

    w_per_i = [_wgen(d2p, w1tp[i], mlp_w2[i], b2r[i]) for i in range(NI)]

    zer = jnp.zeros((N, HID), jnp.float32)
    x = h
    xi = _mm(x, lin1_w[0])
    for i in range(NI):
        parts = _b_calls[i](xi, w_per_i[i], row4d, col4d, zer)
        l1n = lin1_w[(i + 1) % NI]
        x, xi = _c(parts, x, lin2_w[i], lin2_b[i][None, :],
                   lin_w[i], lin_b[i][None, :], l1n)
    return (x, edge_index)


# speedup vs baseline: 1.0018x; 1.0018x over previous
import functools

import numpy as np
import jax
import jax.numpy as jnp
from jax import lax
from jax.experimental import pallas as pl
from jax.experimental.pallas import tpu as pltpu
from jax.experimental.pallas import tpu_sc as plsc

N = 10000
E = 320000
HID = 128
NF = 128
NI = 6
NG = 50
NGP = 64
CUTOFF = 10.0
LOG2 = float(np.log(2.0))
DELTA = CUTOFF / (NG - 1)
COEFF = -0.5 / DELTA**2

NCORES = 2
NSUB = 16
NW = NCORES * NSUB
EPW = E // NW
K = 40
NB = EPW // K
EP = 320512
NCH = 5
CHB = NB // NCH
WCH = 632
WCH_LAST = N - (NSUB - 1) * WCH

_sc_mesh = plsc.VectorSubcoreMesh(core_axis_name="c", subcore_axis_name="s")
_sc_params = pltpu.CompilerParams(needs_layout_passes=False)


def _ssp(x):
    return jnp.maximum(x, 0.0) + jnp.log(1.0 + jnp.exp(-jnp.abs(x))) - LOG2


@functools.partial(
    pl.kernel,
    mesh=_sc_mesh,
    compiler_params=_sc_params,
    out_type=jax.ShapeDtypeStruct((NW, NB, K), jnp.float32),
    scratch_types=[
        pltpu.VMEM((N,), jnp.float32),
        pltpu.VMEM((N,), jnp.float32),
        pltpu.VMEM((N,), jnp.float32),
        pltpu.VMEM((NB, K), jnp.int32),
        pltpu.VMEM((NB, K), jnp.int32),
        pltpu.VMEM((NB, K), jnp.float32),
    ],
)
def _d2(posx, posy, posz, row3d, col3d, d2_out, px, py, pz, ridx, cidx, d2v):
    cid = lax.axis_index("c")
    sid = lax.axis_index("s")
    wid = cid * NSUB + sid
    pltpu.sync_copy(posx, px)
    pltpu.sync_copy(posy, py)
    pltpu.sync_copy(posz, pz)
    pltpu.sync_copy(row3d.at[wid], ridx)
    pltpu.sync_copy(col3d.at[wid], cidx)

    @pl.loop(0, NB)
    def _row(j):
        for k in range(K // 16):
            sl = pl.ds(k * 16, 16)
            r = ridx[j, sl]
            c = cidx[j, sl]
            dx = plsc.load_gather(px, [r]) - plsc.load_gather(px, [c])
            dy = plsc.load_gather(py, [r]) - plsc.load_gather(py, [c])
            dz = plsc.load_gather(pz, [r]) - plsc.load_gather(pz, [c])
            d2v[j, sl] = dx * dx + dy * dy + dz * dz

    pltpu.sync_copy(d2v, d2_out.at[wid])


_BR = 8
_EBLK = _BR * 128


def _wgen_body(d2_ref, w1_ref, w2_ref, b2_ref, out_ref):
    d2 = d2_ref[...]
    d = jnp.sqrt(d2)
    cm = 0.5 * (jnp.cos(d * (np.pi / CUTOFF)) + 1.0)
    w1 = w1_ref[...]
    w2 = w2_ref[...]
    b2 = b2_ref[...]
    g_iota = lax.broadcasted_iota(jnp.int32, (NGP, 128), 0)
    offs = g_iota.astype(jnp.float32) * DELTA
    is_last = g_iota == NGP - 1
    for sb in range(_BR):
        drow = d[sb:sb + 1, :]
        crow = cm[sb:sb + 1, :]
        ea = jnp.exp(COEFF * (drow - offs) ** 2)
        ea = jnp.where(is_last, 1.0, ea)
        h1 = jnp.dot(w1, ea, preferred_element_type=jnp.float32)
        h1 = _ssp(h1) * crow
        w = lax.dot_general(h1, w2, (((0,), (0,)), ((), ())),
                            preferred_element_type=jnp.float32)
        out_ref[sb * 128:(sb + 1) * 128, :] = w + b2


def _wgen(d2p, w1tp_i, w2_i, b2r_i):
    nsteps = (EP // 128) // _BR
    return pl.pallas_call(
        _wgen_body,
        grid=(nsteps,),
        in_specs=[
            pl.BlockSpec((_BR, 128), lambda e: (e, 0)),
            pl.BlockSpec((NF, NGP), lambda e: (0, 0)),
            pl.BlockSpec((NF, NF), lambda e: (0, 0)),
            pl.BlockSpec((1, NF), lambda e: (0, 0)),
        ],
        out_specs=pl.BlockSpec((_EBLK, NF), lambda e: (e, 0)),
        out_shape=jax.ShapeDtypeStruct((EP, NF), jnp.float32),
    )(d2p, w1tp_i, w2_i, b2r_i)


def _make_b(i):
    @functools.partial(
        pl.kernel,
        mesh=_sc_mesh,
        compiler_params=_sc_params,
        out_type=jax.ShapeDtypeStruct((NCORES, N, HID), jnp.float32),
        scratch_types=[
            pltpu.VMEM((CHB, K), jnp.int32),
            pltpu.VMEM((CHB, K), jnp.int32),
            pltpu.VMEM((K, HID), jnp.float32),
            pltpu.VMEM((K, HID), jnp.float32),
            pltpu.VMEM((K, HID), jnp.float32),
            pltpu.VMEM((K, HID), jnp.float32),
            pltpu.VMEM((K, HID), jnp.float32),
            pltpu.VMEM((K, HID), jnp.float32),
            pltpu.VMEM_SHARED((N, HID), jnp.float32),
            pltpu.SemaphoreType.DMA,
            pltpu.SemaphoreType.DMA,
            pltpu.SemaphoreType.DMA,
            pltpu.SemaphoreType.DMA,
            pltpu.SemaphoreType.DMA,
            pltpu.SemaphoreType.DMA,
            pltpu.SemaphoreType.DMA,
            pltpu.SemaphoreType.DMA,
            pltpu.SemaphoreType.DMA,
        ],
    )
    def _b(xi, w_i, row4d, col4d, zer, out, ridx, cidx,
           rows0, w0, rows1, w1, rows2, w2, agg,
           sg0, sg1, sg2, sw0, sw1, sw2, ss0, ss1, ss2):
        cid = lax.axis_index("c")
        sid = lax.axis_index("s")
        wid = cid * NSUB + sid
        r0 = sid * WCH

        def _agg_chunk_copy(src_of, dst_of):
            @pl.when(sid < NSUB - 1)
            def _a():
                pltpu.sync_copy(src_of(pl.ds(r0, WCH)), dst_of(pl.ds(r0, WCH)))

            @pl.when(sid == NSUB - 1)
            def _c():
                pltpu.sync_copy(src_of(pl.ds(r0, WCH_LAST)),
                                dst_of(pl.ds(r0, WCH_LAST)))

        _agg_chunk_copy(lambda s: zer.at[s], lambda s: agg.at[s])
        plsc.subcore_barrier()
        ebase = wid * EPW

        def _mul(rows_, wv_):
            @pl.loop(0, K)
            def _mj(j):
                for v in range(HID // 16):
                    sl = pl.ds(v * 16, 16)
                    rows_[j, sl] = rows_[j, sl] * wv_[j, sl]

        slots = ((rows0, w0, sg0, sw0, ss0),
                 (rows1, w1, sg1, sw1, ss1),
                 (rows2, w2, sg2, sw2, ss2))

        def _drain(sem, buf):
            pltpu.make_async_copy(zer.at[pl.ds(0, K)], buf, sem).wait()

        def _process(bb, s):
            rows_, w_, sg_, sw_, ss_ = slots[s]
            _drain(sg_, rows_)
            _drain(sw_, w_)
            _mul(rows_, w_)
            pltpu.async_copy(rows_, agg.at[cidx.at[bb]], ss_, add=True)

        @pl.loop(0, NCH)
        def _chunk(c):
            base = c * CHB

            def _issue(bb, s):
                rows_, w_, sg_, sw_, _ = slots[s]
                pltpu.async_copy(xi.at[ridx.at[bb]], rows_, sg_)
                pltpu.async_copy(
                    w_i.at[pl.ds(ebase + (base + bb) * K, K)], w_, sw_)

            @pl.when(c > 0)
            def _dc():
                _drain(ss0, rows0)
                _drain(ss1, rows1)

            pltpu.sync_copy(row4d.at[wid, c], ridx)
            pltpu.sync_copy(col4d.at[wid, c], cidx)
            _issue(0, 0)
            _issue(1, 1)

            @pl.loop(0, (CHB - 2) // 3)
            def _triple(t):
                bb = 3 * t

                @pl.when(c + t > 0)
                def _d2s():
                    _drain(ss2, rows2)

                _issue(bb + 2, 2)
                _process(bb, 0)
                _process(bb + 1, 1)
                _drain(ss0, rows0)
                _issue(bb + 3, 0)
                _process(bb + 2, 2)
                _drain(ss1, rows1)
                _issue(bb + 4, 1)

            _process(CHB - 2, 0)
            _process(CHB - 1, 1)

        _drain(ss0, rows0)
        _drain(ss1, rows1)
        _drain(ss2, rows2)

        plsc.subcore_barrier()
        _agg_chunk_copy(lambda s: agg.at[s], lambda s: out.at[cid, s])

    return _b


_b_calls = [_make_b(i) for i in range(NI)]


_BN = 1000


def _mm_body(a_ref, b_ref, o_ref):
    o_ref[...] = jnp.dot(a_ref[...], b_ref[...],
                         preferred_element_type=jnp.float32)


def _mm(a, b):
    return pl.pallas_call(
        _mm_body,
        grid=(N // _BN,),
        in_specs=[
            pl.BlockSpec((_BN, HID), lambda n: (n, 0)),
            pl.BlockSpec((HID, NF), lambda n: (0, 0)),
        ],
        out_specs=pl.BlockSpec((_BN, NF), lambda n: (n, 0)),
        out_shape=jax.ShapeDtypeStruct((N, NF), jnp.float32),
    )(a, b)


def _c_body(parts_ref, x_ref, l2w_ref, l2b_ref, lw_ref, lb_ref, l1n_ref,
            xn_ref, xin_ref):
    agg = parts_ref[0] + parts_ref[1]
    t = jnp.dot(agg, l2w_ref[...], preferred_element_type=jnp.float32)
    t = _ssp(t + l2b_ref[...])
    t = jnp.dot(t, lw_ref[...], preferred_element_type=jnp.float32)
    xn = x_ref[...] + t + lb_ref[...]
    xn_ref[...] = xn
    xin_ref[...] = jnp.dot(xn, l1n_ref[...],
                           preferred_element_type=jnp.float32)


def _c(parts, x, l2w, l2b, lw, lb, l1n):
    return pl.pallas_call(
        _c_body,
        grid=(N // _BN,),
        in_specs=[
            pl.BlockSpec((NCORES, _BN, HID), lambda n: (0, n, 0)),
            pl.BlockSpec((_BN, HID), lambda n: (n, 0)),
            pl.BlockSpec((HID, NF), lambda n: (0, 0)),
            pl.BlockSpec((1, NF), lambda n: (0, 0)),
            pl.BlockSpec((NF, HID), lambda n: (0, 0)),
            pl.BlockSpec((1, HID), lambda n: (0, 0)),
            pl.BlockSpec((HID, NF), lambda n: (0, 0)),
        ],
        out_specs=[
            pl.BlockSpec((_BN, HID), lambda n: (n, 0)),
            pl.BlockSpec((_BN, NF), lambda n: (n, 0)),
        ],
        out_shape=[
            jax.ShapeDtypeStruct((N, HID), jnp.float32),
            jax.ShapeDtypeStruct((N, NF), jnp.float32),
        ],
    )(parts, x, l2w, l2b, lw, lb, l1n)


def kernel(h, pos, edge_index, edge_attr, mlp_w1, mlp_b1, mlp_w2, mlp_b2,
           lin1_w, lin2_w, lin2_b, lin_w, lin_b):
    row3d = edge_index[0].reshape(NW, NB, K)
    col3d = edge_index[1].reshape(NW, NB, K)
    row4d = edge_index[0].reshape(NW, NCH, CHB, K)
    col4d = edge_index[1].reshape(NW, NCH, CHB, K)
    posT = pos.T

    d2_3d = _d2(posT[0], posT[1], posT[2], row3d, col3d)
    d2p = jnp.pad(d2_3d.reshape(E), (0, EP - E)).reshape(EP // 128, 128)

    w1t = jnp.transpose(mlp_w1, (0, 2, 1))
    w1tp = jnp.zeros((NI, NF, NGP), jnp.float32)
    w1tp = w1tp.at[:, :, :NG].set(w1t).at[:, :, NGP - 1].set(mlp_b1)
    b2r = mlp_b2[:, None, :]

    w_per_i = [_wgen(d2p, w1tp[i], mlp_w2[i], b2r[i]) for i in range(NI)]

    zer = jnp.zeros((N, HID), jnp.float32)
    x = h
    xi = _mm(x, lin1_w[0])
    for i in range(NI):
        parts = _b_calls[i](xi, w_per_i[i], row4d, col4d, zer)
        l1n = lin1_w[(i + 1) % NI]
        x, xi = _c(parts, x, lin2_w[i], lin2_b[i][None, :],
                   lin_w[i], lin_b[i][None, :], l1n)
    return (x, edge_index)
